# baseline (speedup 1.0000x reference)
import functools

import jax
import jax.numpy as jnp
from jax import lax
from jax.experimental import pallas as pl
from jax.experimental.pallas import tpu as pltpu
from jax.experimental.pallas import tpu_sc as plsc

LANES = 16
NC = 2
NS = 16
NW = NC * NS
CHUNK = 128


@functools.lru_cache(maxsize=None)
def _build(batch, dim):
    bpw = batch // NW
    nchunk = bpw // CHUNK
    vpr = dim // LANES

    mesh = plsc.VectorSubcoreMesh(core_axis_name="c", subcore_axis_name="s")

    @functools.partial(
        pl.kernel,
        out_type=jax.ShapeDtypeStruct((batch,), jnp.float32),
        mesh=mesh,
        compiler_params=pltpu.CompilerParams(use_tc_tiling_on_sc=False),
        scratch_types=[
            pltpu.VMEM((nchunk, CHUNK), jnp.int32),
            pltpu.VMEM((nchunk, CHUNK), jnp.int32),
            pltpu.VMEM((2, CHUNK, dim), jnp.float32),
            pltpu.VMEM((2, CHUNK, dim), jnp.float32),
            pltpu.VMEM((bpw,), jnp.float32),
            pltpu.SemaphoreType.DMA,
            pltpu.SemaphoreType.DMA,
        ],
    )
    def two_tower(uids_hbm, iids_hbm, utab_hbm, itab_hbm, out_hbm,
                  uidx_v, iidx_v, ubuf_v, ibuf_v, out_v, sem0, sem1):
        wid = lax.axis_index("s") * NC + lax.axis_index("c")
        base = wid * bpw

        pltpu.sync_copy(uids_hbm.at[wid], uidx_v)
        pltpu.sync_copy(iids_hbm.at[wid], iidx_v)

        sems = (sem0, sem1)

        def fire(j):
            b = j % 2
            return (
                pltpu.async_copy(utab_hbm.at[uidx_v.at[j]], ubuf_v.at[b], sems[b]),
                pltpu.async_copy(itab_hbm.at[iidx_v.at[j]], ibuf_v.at[b], sems[b]),
            )

        inflight = fire(0)
        for j in range(nchunk):
            cur = inflight
            if j + 1 < nchunk:
                inflight = fire(j + 1)
            cur[0].wait()
            cur[1].wait()

            uref = ubuf_v.at[j % 2]
            iref = ibuf_v.at[j % 2]
            lane = lax.iota(jnp.int32, LANES)

            def block(b2, _):
                row0 = b2 * LANES
                acc = jnp.zeros((LANES,), jnp.float32)
                for rr in range(LANES):
                    r = row0 + rr
                    s = None
                    for k in range(vpr):
                        uu = uref[r, pl.ds(k * LANES, LANES)]
                        vv = iref[r, pl.ds(k * LANES, LANES)]
                        p = uu * vv
                        s = p if s is None else s + p
                    for h in (1, 2, 4, 8):
                        s = s + s.at[lane ^ h].get(mode="promise_in_bounds")
                    acc = jnp.where(lane == rr, s, acc)
                out_v[pl.ds(j * CHUNK + row0, LANES)] = acc
                return 0

            lax.fori_loop(0, CHUNK // LANES, block, 0)

        pltpu.sync_copy(out_v, out_hbm.at[pl.ds(base, bpw)])

    return two_tower


def kernel(user_ids, item_ids, user_table, item_table):
    batch = user_ids.shape[0]
    dim = user_table.shape[1]
    bpw = batch // NW
    nchunk = bpw // CHUNK
    uids = jnp.asarray(user_ids, jnp.int32).reshape(NW, nchunk, CHUNK)
    iids = jnp.asarray(item_ids, jnp.int32).reshape(NW, nchunk, CHUNK)
    fn = _build(batch, dim)
    return fn(uids, iids, user_table, item_table)
